# baseline (speedup 1.0000x reference)
import functools

import jax
import jax.numpy as jnp
from jax import lax
from jax.experimental import pallas as pl
from jax.experimental.pallas import tpu as pltpu
from jax.experimental.pallas import tpu_sc as plsc

NC = 2
NS = 16
NW = NC * NS
SPC = 2
K = 4


def _sc_gather(table, idx3, bsz, seq, d):
    mesh = plsc.VectorSubcoreMesh(core_axis_name="c", subcore_axis_name="s")
    ch = SPC * seq
    nseqw = bsz // NW
    nchunk = nseqw // SPC
    ngroups = nchunk // K

    @functools.partial(
        pl.kernel,
        mesh=mesh,
        out_type=jax.ShapeDtypeStruct((bsz, seq, d), jnp.float32),
        scratch_types=[
            pltpu.VMEM((nchunk, ch), jnp.int32),
            pltpu.VMEM((K, ch, d), jnp.float32),
            pltpu.VMEM((K, ch, d), jnp.float32),
            pltpu.SemaphoreType.DMA,
            pltpu.SemaphoreType.DMA,
        ],
        compiler_params=pltpu.CompilerParams(use_tc_tiling_on_sc=False),
    )
    def k(table_hbm, idx_hbm, out3_hbm, idx_v, buf_a, buf_b, gsem, ssem):
        wid = lax.axis_index("s") * NC + lax.axis_index("c")
        pltpu.sync_copy(idx_hbm.at[wid], idx_v)
        seq_base = wid * nseqw

        def gather(j, buf, b):
            pltpu.async_copy(table_hbm.at[idx_v.at[j]], buf.at[b], gsem)

        def wait_gather(buf, b):
            pltpu.make_async_copy(table_hbm.at[idx_v.at[0]], buf.at[b], gsem).wait()

        def store(j, buf, b):
            for t in range(SPC):
                pltpu.async_copy(
                    buf.at[b, pl.ds(t * seq, seq)],
                    out3_hbm.at[seq_base + j * SPC + t],
                    ssem,
                )

        def wait_store(buf, b):
            for t in range(SPC):
                pltpu.make_async_copy(
                    buf.at[b, pl.ds(t * seq, seq)], out3_hbm.at[0], ssem
                ).wait()

        for b in range(K):
            gather(b, buf_a, b)

        def half(g, cur, nxt):
            for b in range(K):
                wait_gather(cur, b)

            @pl.when(g + 1 < ngroups)
            def _():
                for b in range(K):
                    gather((g + 1) * K + b, nxt, b)

            for b in range(K):
                store(g * K + b, cur, b)
            for b in range(K):
                wait_store(cur, b)

        def body(t, carry):
            half(2 * t, buf_a, buf_b)
            half(2 * t + 1, buf_b, buf_a)
            return carry

        lax.fori_loop(0, ngroups // 2, body, 0)

    return k(table, idx3)


def kernel(input_ids, word_embeddings):
    bsz, seq = input_ids.shape
    _, d = word_embeddings.shape
    nchunk = bsz // (NW * SPC)
    idx3 = input_ids.reshape(NW, nchunk, SPC * seq).astype(jnp.int32)
    return _sc_gather(word_embeddings, idx3, bsz, seq, d)


# speedup vs baseline: 1.0056x; 1.0056x over previous
import functools

import jax
import jax.numpy as jnp
from jax import lax
from jax.experimental import pallas as pl
from jax.experimental.pallas import tpu as pltpu
from jax.experimental.pallas import tpu_sc as plsc

NC = 2
NS = 16
NW = NC * NS
CB = 128
K = 4


def _sc_gather(table, ids_t, bsz, seq, d):
    mesh = plsc.VectorSubcoreMesh(core_axis_name="c", subcore_axis_name="s")
    bw = bsz // NW
    nbc = bw // CB
    nchunk = seq * nbc
    ngroups = nchunk // K

    @functools.partial(
        pl.kernel,
        mesh=mesh,
        out_type=jax.ShapeDtypeStruct((bsz, seq, d), jnp.float32),
        scratch_types=[
            pltpu.VMEM((seq, bw), jnp.int32),
            pltpu.VMEM((K, CB, d), jnp.float32),
            pltpu.VMEM((K, CB, d), jnp.float32),
            pltpu.SemaphoreType.DMA,
            pltpu.SemaphoreType.DMA,
        ],
        compiler_params=pltpu.CompilerParams(use_tc_tiling_on_sc=False),
    )
    def k(table_hbm, ids_hbm, out3_hbm, idx_v, buf_a, buf_b, gsem, ssem):
        wid = lax.axis_index("s") * NC + lax.axis_index("c")
        bbase = wid * bw
        pltpu.sync_copy(ids_hbm.at[:, pl.ds(bbase, bw)], idx_v)

        def gather(j, buf, b):
            s = j // nbc
            b0 = (j - s * nbc) * CB
            pltpu.async_copy(table_hbm.at[idx_v.at[s, pl.ds(b0, CB)]], buf.at[b], gsem)

        def wait_gather(buf, b):
            pltpu.make_async_copy(
                table_hbm.at[idx_v.at[0, pl.ds(0, CB)]], buf.at[b], gsem
            ).wait()

        def store(j, buf, b):
            s = j // nbc
            b0 = (j - s * nbc) * CB
            pltpu.async_copy(
                buf.at[b], out3_hbm.at[pl.ds(bbase + b0, CB), s], ssem
            )

        def wait_store(buf, b):
            pltpu.make_async_copy(
                buf.at[b], out3_hbm.at[pl.ds(bbase, CB), 0], ssem
            ).wait()

        for b in range(K):
            gather(b, buf_a, b)

        def half(g, cur, nxt):
            for b in range(K):
                wait_gather(cur, b)

            @pl.when(g + 1 < ngroups)
            def _():
                for b in range(K):
                    gather((g + 1) * K + b, nxt, b)

            for b in range(K):
                store(g * K + b, cur, b)
            for b in range(K):
                wait_store(cur, b)

        def body(t, carry):
            half(2 * t, buf_a, buf_b)
            half(2 * t + 1, buf_b, buf_a)
            return carry

        lax.fori_loop(0, ngroups // 2, body, 0)

    return k(table, ids_t)


def kernel(input_ids, word_embeddings):
    bsz, seq = input_ids.shape
    _, d = word_embeddings.shape
    ids_t = input_ids.T.astype(jnp.int32)
    return _sc_gather(word_embeddings, ids_t, bsz, seq, d)
